# baseline (speedup 1.0000x reference)
import functools

import jax
import jax.numpy as jnp
from jax import lax
from jax.experimental import pallas as pl
from jax.experimental.pallas import tpu as pltpu
from jax.experimental.pallas import tpu_sc as plsc

N = 10000
E = 320000
D = 128
D_OUT = 40

NC = 2
NS = 16
NW = NC * NS
EPW = E // NW
CHUNK = 100
NCHUNK = EPW // CHUNK
NBUF = 2
NPAD = 10240
ROWS_PER_TILE = NPAD // NS

_mesh = plsc.VectorSubcoreMesh(
    core_axis_name="c", subcore_axis_name="s", num_cores=NC, num_subcores=NS)


@functools.partial(
    pl.kernel,
    out_type=jax.ShapeDtypeStruct((NC, NPAD, D), jnp.float32),
    mesh=_mesh,
    scratch_types=[
        pltpu.VMEM((NCHUNK, CHUNK), jnp.int32),
        pltpu.VMEM((CHUNK, D), jnp.float32),
        pltpu.VMEM_SHARED((NPAD, D), jnp.float32),
        pltpu.SemaphoreType.DMA,
    ],
)
def _sc_degree(dst_hbm, ones_hbm, z_hbm, out_hbm, dstv, onesv, acc_sh, ssem):
    cid = lax.axis_index("c")
    sid = lax.axis_index("s")
    wid = cid * NS + sid
    pltpu.sync_copy(dst_hbm.at[wid], dstv)
    pltpu.sync_copy(ones_hbm, onesv)
    base = sid * ROWS_PER_TILE
    pltpu.sync_copy(z_hbm.at[pl.ds(base, ROWS_PER_TILE)],
                    acc_sh.at[pl.ds(base, ROWS_PER_TILE)])
    plsc.subcore_barrier()

    def body(c, _):
        pltpu.async_copy(onesv, acc_sh.at[dstv.at[c]], ssem, add=True)
        return _

    lax.fori_loop(0, NCHUNK, body, None)

    def drain(c, _):
        pltpu.make_async_copy(onesv, acc_sh.at[dstv.at[c]], ssem).wait()
        return _

    lax.fori_loop(0, NCHUNK, drain, None)
    plsc.subcore_barrier()
    pltpu.sync_copy(acc_sh.at[pl.ds(base, ROWS_PER_TILE)],
                    out_hbm.at[cid, pl.ds(base, ROWS_PER_TILE)])


@functools.partial(
    pl.kernel,
    out_type=jax.ShapeDtypeStruct((NC, NPAD, D), jnp.float32),
    mesh=_mesh,
    scratch_types=[
        pltpu.VMEM((NCHUNK // 2, CHUNK), jnp.int32),
        pltpu.VMEM((NCHUNK // 2, CHUNK), jnp.int32),
        pltpu.VMEM((NBUF, CHUNK, D), jnp.float32),
        pltpu.VMEM_SHARED((NPAD, D), jnp.float32),
        pltpu.SemaphoreType.DMA,
        pltpu.SemaphoreType.DMA,
    ],
)
def _sc_aggregate(s_hbm, src_hbm, dst_hbm, z_hbm, out_hbm,
                  srcv, dstv, rows, acc_sh, gsem0, gsem1):
    cid = lax.axis_index("c")
    sid = lax.axis_index("s")
    wid = cid * NS + sid
    base = sid * ROWS_PER_TILE
    pltpu.sync_copy(z_hbm.at[pl.ds(base, ROWS_PER_TILE)],
                    acc_sh.at[pl.ds(base, ROWS_PER_TILE)])
    plsc.subcore_barrier()

    sems = (gsem0, gsem1)
    half = NCHUNK // 2

    def body(g, _):
        for b in range(NBUF):
            c = g * NBUF + b
            pltpu.make_async_copy(s_hbm.at[srcv.at[c]], rows.at[b],
                                  sems[b]).wait()
            pltpu.sync_copy(rows.at[b], acc_sh.at[dstv.at[c]], add=True)
            pltpu.async_copy(s_hbm.at[srcv.at[c + NBUF]], rows.at[b],
                             sems[b])
        return _

    for h in range(2):
        pltpu.sync_copy(src_hbm.at[wid, h], srcv)
        pltpu.sync_copy(dst_hbm.at[wid, h], dstv)
        for b in range(NBUF):
            pltpu.async_copy(s_hbm.at[srcv.at[b]], rows.at[b], sems[b])
        lax.fori_loop(0, half // NBUF - 1, body, None)
        for b in range(NBUF):
            c = half - NBUF + b
            pltpu.make_async_copy(s_hbm.at[srcv.at[c]], rows.at[b],
                                  sems[b]).wait()
            pltpu.sync_copy(rows.at[b], acc_sh.at[dstv.at[c]], add=True)
    plsc.subcore_barrier()
    pltpu.sync_copy(acc_sh.at[pl.ds(base, ROWS_PER_TILE)],
                    out_hbm.at[cid, pl.ds(base, ROWS_PER_TILE)])


_RB = 1024


def _dinv_of(deg_ref):
    deg = deg_ref[0, :, 0:1] + deg_ref[1, :, 0:1] + 1.0
    return 1.0 / jnp.sqrt(deg)


def _tc_matmul_body(x_ref, w_ref, o_ref):
    o_ref[...] = jnp.dot(x_ref[...], w_ref[...],
                         preferred_element_type=jnp.float32)


def _tc_scale_body(deg_ref, u_ref, o_ref):
    o_ref[...] = u_ref[...] * _dinv_of(deg_ref)


def _tc_mid_body(deg_ref, acc_ref, s_ref, b_ref, w_ref, o_ref):
    dinv = _dinv_of(deg_ref)
    h = acc_ref[0] + acc_ref[1] + s_ref[...]
    a = jnp.maximum(h * dinv + b_ref[...][None, :], 0.0)
    o_ref[...] = jnp.dot(a, w_ref[...],
                         preferred_element_type=jnp.float32) * dinv


def _tc_last_body(deg_ref, acc_ref, s_ref, b_ref, w_ref, bc_ref, o_ref):
    dinv = _dinv_of(deg_ref)
    h = acc_ref[0] + acc_ref[1] + s_ref[...]
    a = jnp.maximum(h * dinv + b_ref[...][None, :], 0.0)
    o_ref[...] = jnp.dot(a, w_ref[...],
                         preferred_element_type=jnp.float32) + bc_ref[...][None, :]


def _row_spec(d):
    return pl.BlockSpec((_RB, d), lambda i: (i, 0))


_deg_spec = pl.BlockSpec((NC, _RB, D), lambda i: (0, i, 0))
_acc_spec = pl.BlockSpec((NC, _RB, D), lambda i: (0, i, 0))


def _full(shape):
    nd = len(shape)
    return pl.BlockSpec(shape, lambda i: (0,) * nd)


_GRID = (NPAD // _RB,)

_tc_matmul = pl.pallas_call(
    _tc_matmul_body,
    grid=_GRID,
    in_specs=[_row_spec(D), _full((D, D))],
    out_specs=_row_spec(D),
    out_shape=jax.ShapeDtypeStruct((NPAD, D), jnp.float32),
)

_tc_scale = pl.pallas_call(
    _tc_scale_body,
    grid=_GRID,
    in_specs=[_deg_spec, _row_spec(D)],
    out_specs=_row_spec(D),
    out_shape=jax.ShapeDtypeStruct((NPAD, D), jnp.float32),
)

_tc_mid = pl.pallas_call(
    _tc_mid_body,
    grid=_GRID,
    in_specs=[_deg_spec, _acc_spec, _row_spec(D), _full((D,)), _full((D, D))],
    out_specs=_row_spec(D),
    out_shape=jax.ShapeDtypeStruct((NPAD, D), jnp.float32),
)

_tc_last = pl.pallas_call(
    _tc_last_body,
    grid=_GRID,
    in_specs=[_deg_spec, _acc_spec, _row_spec(D), _full((D,)),
              _full((D, D_OUT)), _full((D_OUT,))],
    out_specs=_row_spec(D_OUT),
    out_shape=jax.ShapeDtypeStruct((NPAD, D_OUT), jnp.float32),
)


def kernel(x, edge_index, W1, b1, W2, b2, Wc, bc):
    x = jnp.pad(x, ((0, NPAD - N), (0, 0)))
    ei = edge_index.astype(jnp.int32)
    src4 = ei[0].reshape(NW, 2, NCHUNK // 2, CHUNK)
    dst4 = ei[1].reshape(NW, 2, NCHUNK // 2, CHUNK)
    dst3 = ei[1].reshape(NW, NCHUNK, CHUNK)
    ones128 = jnp.ones((CHUNK, D), jnp.float32)
    z128 = jnp.zeros((NPAD, D), jnp.float32)

    deg16 = _sc_degree(dst3, ones128, z128)
    u1 = _tc_matmul(x, W1)
    s1 = _tc_scale(deg16, u1)
    acc1 = _sc_aggregate(s1, src4, dst4, z128)
    s2 = _tc_mid(deg16, acc1, s1, b1, W2)
    acc2 = _sc_aggregate(s2, src4, dst4, z128)
    return _tc_last(deg16, acc2, s2, b2, Wc, bc)[:N]


# speedup vs baseline: 1.0190x; 1.0190x over previous
import functools

import jax
import jax.numpy as jnp
from jax import lax
from jax.experimental import pallas as pl
from jax.experimental.pallas import tpu as pltpu
from jax.experimental.pallas import tpu_sc as plsc

N = 10000
E = 320000
D = 128
D_OUT = 40

NC = 2
NS = 16
NW = NC * NS
EPW = E // NW
CHUNK = 125
NCHUNK = EPW // CHUNK
NBUF = 2
NPAD = 10240
ROWS_PER_TILE = NPAD // NS

_mesh = plsc.VectorSubcoreMesh(
    core_axis_name="c", subcore_axis_name="s", num_cores=NC, num_subcores=NS)


@functools.partial(
    pl.kernel,
    out_type=jax.ShapeDtypeStruct((NC, NPAD, D), jnp.float32),
    mesh=_mesh,
    scratch_types=[
        pltpu.VMEM((NCHUNK, CHUNK), jnp.int32),
        pltpu.VMEM((CHUNK, D), jnp.float32),
        pltpu.VMEM_SHARED((NPAD, D), jnp.float32),
        pltpu.SemaphoreType.DMA,
    ],
)
def _sc_degree(dst_hbm, ones_hbm, z_hbm, out_hbm, dstv, onesv, acc_sh, ssem):
    cid = lax.axis_index("c")
    sid = lax.axis_index("s")
    wid = cid * NS + sid
    pltpu.sync_copy(dst_hbm.at[wid], dstv)
    pltpu.sync_copy(ones_hbm, onesv)
    base = sid * ROWS_PER_TILE
    pltpu.sync_copy(z_hbm.at[pl.ds(base, ROWS_PER_TILE)],
                    acc_sh.at[pl.ds(base, ROWS_PER_TILE)])
    plsc.subcore_barrier()

    def body(c, _):
        pltpu.async_copy(onesv, acc_sh.at[dstv.at[c]], ssem, add=True)
        return _

    lax.fori_loop(0, NCHUNK, body, None)

    def drain(c, _):
        pltpu.make_async_copy(onesv, acc_sh.at[dstv.at[c]], ssem).wait()
        return _

    lax.fori_loop(0, NCHUNK, drain, None)
    plsc.subcore_barrier()
    pltpu.sync_copy(acc_sh.at[pl.ds(base, ROWS_PER_TILE)],
                    out_hbm.at[cid, pl.ds(base, ROWS_PER_TILE)])


@functools.partial(
    pl.kernel,
    out_type=jax.ShapeDtypeStruct((NC, NPAD, D), jnp.float32),
    mesh=_mesh,
    scratch_types=[
        pltpu.VMEM((NCHUNK // 2, CHUNK), jnp.int32),
        pltpu.VMEM((NCHUNK // 2, CHUNK), jnp.int32),
        pltpu.VMEM((NBUF, CHUNK, D), jnp.float32),
        pltpu.VMEM_SHARED((NPAD, D), jnp.float32),
        pltpu.SemaphoreType.DMA,
        pltpu.SemaphoreType.DMA,
    ],
)
def _sc_aggregate(s_hbm, src_hbm, dst_hbm, z_hbm, out_hbm,
                  srcv, dstv, rows, acc_sh, gsem0, gsem1):
    cid = lax.axis_index("c")
    sid = lax.axis_index("s")
    wid = cid * NS + sid
    base = sid * ROWS_PER_TILE
    pltpu.sync_copy(z_hbm.at[pl.ds(base, ROWS_PER_TILE)],
                    acc_sh.at[pl.ds(base, ROWS_PER_TILE)])
    plsc.subcore_barrier()

    sems = (gsem0, gsem1)
    half = NCHUNK // 2

    def body(g, _):
        for b in range(NBUF):
            c = g * NBUF + b
            pltpu.make_async_copy(s_hbm.at[srcv.at[c]], rows.at[b],
                                  sems[b]).wait()
            pltpu.sync_copy(rows.at[b], acc_sh.at[dstv.at[c]], add=True)
            pltpu.async_copy(s_hbm.at[srcv.at[c + NBUF]], rows.at[b],
                             sems[b])
        return _

    for h in range(2):
        pltpu.sync_copy(src_hbm.at[wid, h], srcv)
        pltpu.sync_copy(dst_hbm.at[wid, h], dstv)
        for b in range(NBUF):
            pltpu.async_copy(s_hbm.at[srcv.at[b]], rows.at[b], sems[b])
        lax.fori_loop(0, half // NBUF - 1, body, None)
        for b in range(NBUF):
            c = half - NBUF + b
            pltpu.make_async_copy(s_hbm.at[srcv.at[c]], rows.at[b],
                                  sems[b]).wait()
            pltpu.sync_copy(rows.at[b], acc_sh.at[dstv.at[c]], add=True)
    plsc.subcore_barrier()
    pltpu.sync_copy(acc_sh.at[pl.ds(base, ROWS_PER_TILE)],
                    out_hbm.at[cid, pl.ds(base, ROWS_PER_TILE)])


_RB = 1024


def _dinv_of(deg_ref):
    deg = deg_ref[0, :, 0:1] + deg_ref[1, :, 0:1] + 1.0
    return 1.0 / jnp.sqrt(deg)


def _tc_matmul_body(x_ref, w_ref, o_ref):
    o_ref[...] = jnp.dot(x_ref[...], w_ref[...],
                         preferred_element_type=jnp.float32)


def _tc_scale_body(deg_ref, u_ref, o_ref):
    o_ref[...] = u_ref[...] * _dinv_of(deg_ref)


def _tc_mid_body(deg_ref, acc_ref, s_ref, b_ref, w_ref, o_ref):
    dinv = _dinv_of(deg_ref)
    h = acc_ref[0] + acc_ref[1] + s_ref[...]
    a = jnp.maximum(h * dinv + b_ref[...][None, :], 0.0)
    o_ref[...] = jnp.dot(a, w_ref[...],
                         preferred_element_type=jnp.float32) * dinv


def _tc_last_body(deg_ref, acc_ref, s_ref, b_ref, w_ref, bc_ref, o_ref):
    dinv = _dinv_of(deg_ref)
    h = acc_ref[0] + acc_ref[1] + s_ref[...]
    a = jnp.maximum(h * dinv + b_ref[...][None, :], 0.0)
    o_ref[...] = jnp.dot(a, w_ref[...],
                         preferred_element_type=jnp.float32) + bc_ref[...][None, :]


def _row_spec(d):
    return pl.BlockSpec((_RB, d), lambda i: (i, 0))


_deg_spec = pl.BlockSpec((NC, _RB, D), lambda i: (0, i, 0))
_acc_spec = pl.BlockSpec((NC, _RB, D), lambda i: (0, i, 0))


def _full(shape):
    nd = len(shape)
    return pl.BlockSpec(shape, lambda i: (0,) * nd)


_GRID = (NPAD // _RB,)

_tc_matmul = pl.pallas_call(
    _tc_matmul_body,
    grid=_GRID,
    in_specs=[_row_spec(D), _full((D, D))],
    out_specs=_row_spec(D),
    out_shape=jax.ShapeDtypeStruct((NPAD, D), jnp.float32),
)

_tc_scale = pl.pallas_call(
    _tc_scale_body,
    grid=_GRID,
    in_specs=[_deg_spec, _row_spec(D)],
    out_specs=_row_spec(D),
    out_shape=jax.ShapeDtypeStruct((NPAD, D), jnp.float32),
)

_tc_mid = pl.pallas_call(
    _tc_mid_body,
    grid=_GRID,
    in_specs=[_deg_spec, _acc_spec, _row_spec(D), _full((D,)), _full((D, D))],
    out_specs=_row_spec(D),
    out_shape=jax.ShapeDtypeStruct((NPAD, D), jnp.float32),
)

_tc_last = pl.pallas_call(
    _tc_last_body,
    grid=_GRID,
    in_specs=[_deg_spec, _acc_spec, _row_spec(D), _full((D,)),
              _full((D, D_OUT)), _full((D_OUT,))],
    out_specs=_row_spec(D_OUT),
    out_shape=jax.ShapeDtypeStruct((NPAD, D_OUT), jnp.float32),
)


def kernel(x, edge_index, W1, b1, W2, b2, Wc, bc):
    x = jnp.pad(x, ((0, NPAD - N), (0, 0)))
    ei = edge_index.astype(jnp.int32)
    src4 = ei[0].reshape(NW, 2, NCHUNK // 2, CHUNK)
    dst4 = ei[1].reshape(NW, 2, NCHUNK // 2, CHUNK)
    dst3 = ei[1].reshape(NW, NCHUNK, CHUNK)
    ones128 = jnp.ones((CHUNK, D), jnp.float32)
    z128 = jnp.zeros((NPAD, D), jnp.float32)

    deg16 = _sc_degree(dst3, ones128, z128)
    u1 = _tc_matmul(x, W1)
    s1 = _tc_scale(deg16, u1)
    acc1 = _sc_aggregate(s1, src4, dst4, z128)
    s2 = _tc_mid(deg16, acc1, s1, b1, W2)
    acc2 = _sc_aggregate(s2, src4, dst4, z128)
    return _tc_last(deg16, acc2, s2, b2, Wc, bc)[:N]
